# baseline (speedup 1.0000x reference)
import functools

import jax
import jax.numpy as jnp
from jax.experimental import pallas as pl


_EPS = 1e-5


def _dot_t(a, w):
    return jax.lax.dot_general(
        a.astype(jnp.bfloat16), w.astype(jnp.bfloat16),
        (((1,), (1,)), ((), ())), preferred_element_type=jnp.float32
    )


def _bn_affine(st, n_rows, b_prev, g, be):
    s = st[0:1, :]
    q = st[1:2, :]
    m = s * (1.0 / n_rows)
    v = q * (1.0 / n_rows) - m * m
    a = g * jax.lax.rsqrt(v + _EPS)
    o = (b_prev - m) * a + be
    return a, o


def _lrelu(t):
    return jnp.where(t >= 0, t, 0.1 * t)


def _accum_stats(i, y, st_ref):
    s = jnp.sum(y, axis=0, keepdims=True)
    q = jnp.sum(y * y, axis=0, keepdims=True)
    sq = jnp.concatenate([s, q], axis=0)

    @pl.when(i == 0)
    def _():
        st_ref[...] = jnp.zeros_like(st_ref)

    st_ref[...] += sq


def _stats_first_kernel(x_ref, w_ref, b_ref, st_ref):
    i = pl.program_id(0)
    y = _dot_t(x_ref[...], w_ref[...]) + b_ref[...]
    _accum_stats(i, y, st_ref)


def _mid_kernel(xin_ref, st_ref, p_ref, wprev_ref, wnext_ref, xout_ref,
                stout_ref, *, n_rows):
    i = pl.program_id(0)
    x = xin_ref[...].astype(jnp.float32)
    a, o = _bn_affine(st_ref[...], n_rows, p_ref[0:1, :], p_ref[1:2, :],
                      p_ref[2:3, :])
    t = _dot_t(x, wprev_ref[...]) * a + o
    x1 = _lrelu(t) + 0.1 * x
    xout_ref[...] = x1.astype(xout_ref.dtype)
    y2 = _dot_t(x1, wnext_ref[...]) + p_ref[3:4, :]
    _accum_stats(i, y2, stout_ref)


def _final_kernel(xin_ref, st_ref, p_ref, wprev_ref, w4_ref, b4_ref, out_ref,
                  *, n_rows):
    x = xin_ref[...].astype(jnp.float32)
    a, o = _bn_affine(st_ref[...], n_rows, p_ref[0:1, :], p_ref[1:2, :],
                      p_ref[2:3, :])
    t = _dot_t(x, wprev_ref[...]) * a + o
    x3 = _lrelu(t) + 0.1 * x
    o8 = _dot_t(x3, w4_ref[...])
    out_ref[...] = o8[:, 0:1] + b4_ref[0, 0]


def _pick_block(n):
    for bn in (10000, 2000, 1000, 800, 500, 250, 200, 104, 100, 50, 40, 25, 20, 8):
        if n % bn == 0 and bn % 8 == 0:
            return bn
    return n


_PROBE = 0


def kernel(x, W1, b1, g1, be1, W2, b2, g2, be2, W3, b3, g3, be3, W4, b4):
    n, d = x.shape
    bn_rows = _pick_block(n)
    nb = n // bn_rows
    grid = (nb,)

    row = lambda v: v.reshape(1, d)
    p12 = jnp.concatenate([row(b1), row(g1), row(be1), row(b2)], axis=0)
    p23 = jnp.concatenate([row(b2), row(g2), row(be2), row(b3)], axis=0)
    p3 = jnp.concatenate([row(b3), row(g3), row(be3)], axis=0)

    xs = pl.BlockSpec((bn_rows, d), lambda i: (i, 0))
    ws = pl.BlockSpec((d, d), lambda i: (0, 0))
    sts = pl.BlockSpec((2, d), lambda i: (0, 0))
    p4s = pl.BlockSpec((4, d), lambda i: (0, 0))
    p3s = pl.BlockSpec((3, d), lambda i: (0, 0))
    b1s = pl.BlockSpec((1, d), lambda i: (0, 0))

    st_shape = jax.ShapeDtypeStruct((2, d), jnp.float32)
    act_shape = jax.ShapeDtypeStruct((n, d), jnp.bfloat16)

    st1 = pl.pallas_call(
        _stats_first_kernel,
        grid=grid,
        in_specs=[xs, ws, b1s],
        out_specs=sts,
        out_shape=st_shape,
    )(x, W1, row(b1))
    if _PROBE == 1:
        return jnp.broadcast_to(st1[0:1, 0:1], (n, 1))

    x1, st2 = pl.pallas_call(
        functools.partial(_mid_kernel, n_rows=float(n)),
        grid=grid,
        in_specs=[xs, sts, p4s, ws, ws],
        out_specs=[xs, sts],
        out_shape=[act_shape, st_shape],
    )(x, st1, p12, W1, W2)
    if _PROBE == 2:
        return jnp.broadcast_to(st2[0:1, 0:1], (n, 1))

    x2, st3 = pl.pallas_call(
        functools.partial(_mid_kernel, n_rows=float(n)),
        grid=grid,
        in_specs=[xs, sts, p4s, ws, ws],
        out_specs=[xs, sts],
        out_shape=[act_shape, st_shape],
    )(x1, st2, p23, W2, W3)

    w4p = jnp.concatenate([W4, jnp.zeros((7, d), jnp.float32)], axis=0)
    out = pl.pallas_call(
        functools.partial(_final_kernel, n_rows=float(n)),
        grid=grid,
        in_specs=[xs, sts, p3s, ws, pl.BlockSpec((8, d), lambda i: (0, 0)),
                  pl.BlockSpec((1, 1), lambda i: (0, 0))],
        out_specs=pl.BlockSpec((bn_rows, 1), lambda i: (i, 0)),
        out_shape=jax.ShapeDtypeStruct((n, 1), jnp.float32),
    )(x2, st3, p3, W3, w4p, b4.reshape(1, 1))

    return out


# speedup vs baseline: 1.1982x; 1.1982x over previous
import functools

import jax
import jax.numpy as jnp
from jax.experimental import pallas as pl


_EPS = 1e-5


def _dot_t(a, w):
    return jax.lax.dot_general(
        a.astype(jnp.bfloat16), w.astype(jnp.bfloat16),
        (((1,), (1,)), ((), ())), preferred_element_type=jnp.float32
    )


def _bn_affine(st, n_rows, b_prev, g, be):
    s = st[0:1, :]
    q = st[1:2, :]
    m = s * (1.0 / n_rows)
    v = q * (1.0 / n_rows) - m * m
    a = g * jax.lax.rsqrt(v + _EPS)
    o = (b_prev - m) * a + be
    return a, o


def _lrelu(t):
    return jnp.where(t >= 0, t, 0.1 * t)


def _accum_stats(i, y, st_ref):
    s = jnp.sum(y, axis=0, keepdims=True)
    q = jnp.sum(y * y, axis=0, keepdims=True)
    sq = jnp.concatenate([s, q], axis=0)

    @pl.when(i == 0)
    def _():
        st_ref[...] = jnp.zeros_like(st_ref)

    st_ref[...] += sq


def _stats_first_kernel(x_ref, w_ref, b_ref, st_ref):
    i = pl.program_id(0)
    y = _dot_t(x_ref[...], w_ref[...]) + b_ref[...]
    _accum_stats(i, y, st_ref)


def _mid_kernel(xin_ref, st_ref, p_ref, wprev_ref, wnext_ref, xout_ref,
                stout_ref, *, n_rows):
    i = pl.program_id(0)
    x = xin_ref[...].astype(jnp.float32)
    a, o = _bn_affine(st_ref[...], n_rows, p_ref[0:1, :], p_ref[1:2, :],
                      p_ref[2:3, :])
    t = _dot_t(x, wprev_ref[...]) * a + o
    x1 = _lrelu(t) + 0.1 * x
    xout_ref[...] = x1.astype(xout_ref.dtype)
    y2 = _dot_t(x1, wnext_ref[...]) + p_ref[3:4, :]
    _accum_stats(i, y2, stout_ref)


def _final_kernel(xin_ref, st_ref, p_ref, wprev_ref, w4_ref, b4_ref, out_ref,
                  *, n_rows):
    x = xin_ref[...].astype(jnp.float32)
    a, o = _bn_affine(st_ref[...], n_rows, p_ref[0:1, :], p_ref[1:2, :],
                      p_ref[2:3, :])
    t = _dot_t(x, wprev_ref[...]) * a + o
    x3 = _lrelu(t) + 0.1 * x
    o8 = jax.lax.dot_general(
        w4_ref[...].astype(jnp.bfloat16), x3.astype(jnp.bfloat16),
        (((1,), (1,)), ((), ())), preferred_element_type=jnp.float32)
    out_ref[...] = (o8[0:1, :] + b4_ref[0, 0]).reshape(out_ref.shape)


def _pick_block(n):
    for bn in (10000, 2000, 1000, 800, 500, 250, 200, 104, 100, 50, 40, 25, 20, 8):
        if n % bn == 0 and bn % 8 == 0:
            return bn
    return n


_PROBE = 0


def kernel(x, W1, b1, g1, be1, W2, b2, g2, be2, W3, b3, g3, be3, W4, b4):
    n, d = x.shape
    bn_rows = _pick_block(n)
    nb = n // bn_rows
    grid = (nb,)

    row = lambda v: v.reshape(1, d)
    p12 = jnp.concatenate([row(b1), row(g1), row(be1), row(b2)], axis=0)
    p23 = jnp.concatenate([row(b2), row(g2), row(be2), row(b3)], axis=0)
    p3 = jnp.concatenate([row(b3), row(g3), row(be3)], axis=0)

    xs = pl.BlockSpec((bn_rows, d), lambda i: (i, 0))
    ws = pl.BlockSpec((d, d), lambda i: (0, 0))
    sts = pl.BlockSpec((2, d), lambda i: (0, 0))
    p4s = pl.BlockSpec((4, d), lambda i: (0, 0))
    p3s = pl.BlockSpec((3, d), lambda i: (0, 0))
    b1s = pl.BlockSpec((1, d), lambda i: (0, 0))

    st_shape = jax.ShapeDtypeStruct((2, d), jnp.float32)
    act_shape = jax.ShapeDtypeStruct((n, d), jnp.bfloat16)

    st1 = pl.pallas_call(
        _stats_first_kernel,
        grid=grid,
        in_specs=[xs, ws, b1s],
        out_specs=sts,
        out_shape=st_shape,
    )(x, W1, row(b1))
    if _PROBE == 1:
        return jnp.broadcast_to(st1[0:1, 0:1], (n, 1))

    x1, st2 = pl.pallas_call(
        functools.partial(_mid_kernel, n_rows=float(n)),
        grid=grid,
        in_specs=[xs, sts, p4s, ws, ws],
        out_specs=[xs, sts],
        out_shape=[act_shape, st_shape],
    )(x, st1, p12, W1, W2)
    if _PROBE == 2:
        return jnp.broadcast_to(st2[0:1, 0:1], (n, 1))

    x2, st3 = pl.pallas_call(
        functools.partial(_mid_kernel, n_rows=float(n)),
        grid=grid,
        in_specs=[xs, sts, p4s, ws, ws],
        out_specs=[xs, sts],
        out_shape=[act_shape, st_shape],
    )(x1, st2, p23, W2, W3)
    if _PROBE == 3:
        return jnp.broadcast_to(st3[0:1, 0:1], (n, 1))

    w4p = jnp.concatenate([W4, jnp.zeros((7, d), jnp.float32)], axis=0)
    out_t = pl.pallas_call(
        functools.partial(_final_kernel, n_rows=float(n)),
        grid=grid,
        in_specs=[xs, sts, p3s, ws, pl.BlockSpec((8, d), lambda i: (0, 0)),
                  pl.BlockSpec((1, 1), lambda i: (0, 0))],
        out_specs=pl.BlockSpec((1, 1, bn_rows), lambda i: (i, 0, 0)),
        out_shape=jax.ShapeDtypeStruct((nb, 1, bn_rows), jnp.float32),
    )(x2, st3, p3, W3, w4p, b4.reshape(1, 1))

    return out_t.reshape(n, 1)


# speedup vs baseline: 1.2678x; 1.0580x over previous
import functools

import jax
import jax.numpy as jnp
from jax.experimental import pallas as pl
from jax.experimental.pallas import tpu as pltpu


_EPS = 1e-5


def _dot_t(a, w):
    return jax.lax.dot_general(
        a.astype(jnp.bfloat16), w.astype(jnp.bfloat16),
        (((1,), (1,)), ((), ())), preferred_element_type=jnp.float32
    )


def _bn_affine(st, n_rows, b_prev, g, be):
    s = st[0:1, :]
    q = st[1:2, :]
    m = s * (1.0 / n_rows)
    v = q * (1.0 / n_rows) - m * m
    a = g * jax.lax.rsqrt(v + _EPS)
    o = (b_prev - m) * a + be
    return a, o


def _lrelu(t):
    return jnp.maximum(t, 0.1 * t)


def _col_stats(y):
    s = jnp.sum(y, axis=0, keepdims=True)
    q = jnp.sum(y * y, axis=0, keepdims=True)
    return jnp.concatenate([s, q], axis=0)


def _fused_kernel(x_ref, w1_ref, w2_ref, w3_ref, w4_ref, pars_ref, b4_ref,
                  out_ref, act_ref, st_ref, *, n_rows, bn):
    p = pl.program_id(0)
    i = pl.program_id(1)
    rows = pl.ds(i * bn, bn)

    @pl.when(jnp.logical_and(p == 0, i == 0))
    def _():
        st_ref[...] = jnp.zeros_like(st_ref)

    @pl.when(p == 0)
    def _():
        xb = x_ref[...].astype(jnp.bfloat16)
        y1 = _dot_t(xb, w1_ref[...]) + pars_ref[0:1, :]
        st_ref[0:2, :] += _col_stats(y1)

    @pl.when(p == 1)
    def _():
        xb = x_ref[...].astype(jnp.bfloat16)
        a1, o1 = _bn_affine(st_ref[0:2, :], n_rows, pars_ref[0:1, :],
                            pars_ref[1:2, :], pars_ref[2:3, :])
        t = _dot_t(xb, w1_ref[...]) * a1 + o1
        x1 = _lrelu(t) + 0.1 * xb.astype(jnp.float32)
        x1b = x1.astype(jnp.bfloat16)
        act_ref[rows, :] = x1b
        y2 = _dot_t(x1b, w2_ref[...]) + pars_ref[3:4, :]
        st_ref[2:4, :] += _col_stats(y2)

    @pl.when(p == 2)
    def _():
        x1b = act_ref[rows, :]
        a2, o2 = _bn_affine(st_ref[2:4, :], n_rows, pars_ref[3:4, :],
                            pars_ref[4:5, :], pars_ref[5:6, :])
        t = _dot_t(x1b, w2_ref[...]) * a2 + o2
        x2 = _lrelu(t) + 0.1 * x1b.astype(jnp.float32)
        x2b = x2.astype(jnp.bfloat16)
        act_ref[rows, :] = x2b
        y3 = _dot_t(x2b, w3_ref[...]) + pars_ref[6:7, :]
        st_ref[4:6, :] += _col_stats(y3)

    @pl.when(p == 3)
    def _():
        x2b = act_ref[rows, :]
        a3, o3 = _bn_affine(st_ref[4:6, :], n_rows, pars_ref[6:7, :],
                            pars_ref[7:8, :], pars_ref[8:9, :])
        t = _dot_t(x2b, w3_ref[...]) * a3 + o3
        x3 = _lrelu(t) + 0.1 * x2b.astype(jnp.float32)
        o8 = jax.lax.dot_general(
            w4_ref[...].astype(jnp.bfloat16), x3.astype(jnp.bfloat16),
            (((1,), (1,)), ((), ())), preferred_element_type=jnp.float32)
        out_ref[...] = (o8[0:1, :] + b4_ref[0, 0]).reshape(out_ref.shape)


def _pick_block(n):
    for bn in (4000, 2000, 1000, 800, 500, 250, 200, 104, 100, 50, 40, 25,
               20, 8):
        if n % bn == 0 and bn % 8 == 0:
            return bn
    return n


def kernel(x, W1, b1, g1, be1, W2, b2, g2, be2, W3, b3, g3, be3, W4, b4):
    n, d = x.shape
    bn = _pick_block(n)
    nb = n // bn

    row = lambda v: v.reshape(1, d)
    pars = jnp.concatenate(
        [row(b1), row(g1), row(be1),
         row(b2), row(g2), row(be2),
         row(b3), row(g3), row(be3)], axis=0)
    w4p = jnp.concatenate([W4, jnp.zeros((7, d), jnp.float32)], axis=0)

    xs = pl.BlockSpec((bn, d), lambda p, i: (jnp.where(p < 2, i, 0), 0))
    ws = pl.BlockSpec((d, d), lambda p, i: (0, 0))

    out_t = pl.pallas_call(
        functools.partial(_fused_kernel, n_rows=float(n), bn=bn),
        grid=(4, nb),
        in_specs=[
            xs, ws, ws, ws,
            pl.BlockSpec((8, d), lambda p, i: (0, 0)),
            pl.BlockSpec((9, d), lambda p, i: (0, 0)),
            pl.BlockSpec((1, 1), lambda p, i: (0, 0)),
        ],
        out_specs=pl.BlockSpec((1, 1, bn),
                               lambda p, i: (jnp.where(p == 3, i, 0), 0, 0)),
        out_shape=jax.ShapeDtypeStruct((nb, 1, bn), jnp.float32),
        scratch_shapes=[
            pltpu.VMEM((n, d), jnp.bfloat16),
            pltpu.VMEM((8, d), jnp.float32),
        ],
        compiler_params=pltpu.CompilerParams(
            dimension_semantics=("arbitrary", "arbitrary"),
            vmem_limit_bytes=100 * 1024 * 1024,
        ),
    )(x, W1, W2, W3, w4p, pars, b4.reshape(1, 1))

    return out_t.reshape(n, 1)


# speedup vs baseline: 1.2819x; 1.0111x over previous
import functools

import jax
import jax.numpy as jnp
from jax.experimental import pallas as pl
from jax.experimental.pallas import tpu as pltpu


_EPS = 1e-5


def _dot_t(a, w):
    return jax.lax.dot_general(
        a.astype(jnp.bfloat16), w.astype(jnp.bfloat16),
        (((1,), (1,)), ((), ())), preferred_element_type=jnp.float32
    )


def _bn_affine(st, n_rows, b_prev, g, be):
    s = st[0:1, :]
    q = st[1:2, :]
    m = s * (1.0 / n_rows)
    v = q * (1.0 / n_rows) - m * m
    a = g * jax.lax.rsqrt(v + _EPS)
    o = (b_prev - m) * a + be
    return a, o


def _lrelu(t):
    return jnp.maximum(t, 0.1 * t)


def _col_stats(y):
    s = jnp.sum(y, axis=0, keepdims=True)
    q = jnp.sum(y * y, axis=0, keepdims=True)
    return jnp.concatenate([s, q], axis=0)


def _fused_kernel(x_ref, w1_ref, w2_ref, w3_ref, w4_ref, pars_ref, b4_ref,
                  out_ref, act_ref, st_ref, *, n_rows, bn):
    p = pl.program_id(0)
    i = pl.program_id(1)
    rows = pl.ds(i * bn, bn)

    @pl.when(jnp.logical_and(p == 0, i == 0))
    def _():
        st_ref[...] = jnp.zeros_like(st_ref)

    @pl.when(p == 0)
    def _():
        xb = x_ref[...].astype(jnp.bfloat16)
        y1 = _dot_t(xb, w1_ref[...]) + pars_ref[0:1, :]
        st_ref[0:2, :] += _col_stats(y1)

    @pl.when(p == 1)
    def _():
        xb = x_ref[...].astype(jnp.bfloat16)
        a1, o1 = _bn_affine(st_ref[0:2, :], n_rows, pars_ref[0:1, :],
                            pars_ref[1:2, :], pars_ref[2:3, :])
        t = _dot_t(xb, w1_ref[...]) * a1 + o1
        x1 = _lrelu(t) + 0.1 * xb.astype(jnp.float32)
        x1b = x1.astype(jnp.bfloat16)
        act_ref[rows, :] = x1b
        y2 = _dot_t(x1b, w2_ref[...]) + pars_ref[3:4, :]
        st_ref[2:4, :] += _col_stats(y2)

    @pl.when(p == 2)
    def _():
        x1b = act_ref[rows, :]
        a2, o2 = _bn_affine(st_ref[2:4, :], n_rows, pars_ref[3:4, :],
                            pars_ref[4:5, :], pars_ref[5:6, :])
        t = _dot_t(x1b, w2_ref[...]) * a2 + o2
        x2 = _lrelu(t) + 0.1 * x1b.astype(jnp.float32)
        x2b = x2.astype(jnp.bfloat16)
        act_ref[rows, :] = x2b
        y3 = _dot_t(x2b, w3_ref[...]) + pars_ref[6:7, :]
        st_ref[4:6, :] += _col_stats(y3)

    @pl.when(p == 3)
    def _():
        x2b = act_ref[rows, :]
        a3, o3 = _bn_affine(st_ref[4:6, :], n_rows, pars_ref[6:7, :],
                            pars_ref[7:8, :], pars_ref[8:9, :])
        t = _dot_t(x2b, w3_ref[...]) * a3 + o3
        x3 = _lrelu(t) + 0.1 * x2b.astype(jnp.float32)
        o8 = jax.lax.dot_general(
            w4_ref[...].astype(jnp.bfloat16), x3.astype(jnp.bfloat16),
            (((1,), (1,)), ((), ())), preferred_element_type=jnp.float32)
        out_ref[...] = (o8[0:1, :] + b4_ref[0, 0]).reshape(out_ref.shape)


def _pick_block(n):
    for bn in (10000, 4000, 2000, 1000, 800, 500, 250, 200, 104, 100, 50, 40,
               25, 20, 8):
        if n % bn == 0 and bn % 8 == 0:
            return bn
    return n


def kernel(x, W1, b1, g1, be1, W2, b2, g2, be2, W3, b3, g3, be3, W4, b4):
    n, d = x.shape
    bn = _pick_block(n)
    nb = n // bn

    row = lambda v: v.reshape(1, d)
    pars = jnp.concatenate(
        [row(b1), row(g1), row(be1),
         row(b2), row(g2), row(be2),
         row(b3), row(g3), row(be3)], axis=0)
    w4p = jnp.concatenate([W4, jnp.zeros((7, d), jnp.float32)], axis=0)

    xs = pl.BlockSpec((bn, d), lambda p, i: (jnp.where(p < 2, i, 0), 0))
    ws = pl.BlockSpec((d, d), lambda p, i: (0, 0))

    out_t = pl.pallas_call(
        functools.partial(_fused_kernel, n_rows=float(n), bn=bn),
        grid=(4, nb),
        in_specs=[
            xs, ws, ws, ws,
            pl.BlockSpec((8, d), lambda p, i: (0, 0)),
            pl.BlockSpec((9, d), lambda p, i: (0, 0)),
            pl.BlockSpec((1, 1), lambda p, i: (0, 0)),
        ],
        out_specs=pl.BlockSpec((1, 1, bn),
                               lambda p, i: (jnp.where(p == 3, i, 0), 0, 0)),
        out_shape=jax.ShapeDtypeStruct((nb, 1, bn), jnp.float32),
        scratch_shapes=[
            pltpu.VMEM((n, d), jnp.bfloat16),
            pltpu.VMEM((8, d), jnp.float32),
        ],
        compiler_params=pltpu.CompilerParams(
            dimension_semantics=("arbitrary", "arbitrary"),
            vmem_limit_bytes=100 * 1024 * 1024,
        ),
    )(x, W1, W2, W3, w4p, pars, b4.reshape(1, 1))

    return out_t.reshape(n, 1)
